# baseline (speedup 1.0000x reference)
import functools

import jax
import jax.numpy as jnp
from jax import lax
from jax.experimental import pallas as pl
from jax.experimental.pallas import tpu as pltpu
from jax.experimental.pallas import tpu_sc as plsc

NUM_CORES = 2
NUM_SUBCORES = 16
NUM_WORKERS = NUM_CORES * NUM_SUBCORES

IDX_PER_GATHER = 128
GATHERS_PER_BLOCK = 4


def _gather_kernel_body(n_gathers, weight_hbm, idx_hbm, out_hbm,
                        idx_v, rows_v, gsem, osem):
    wid = lax.axis_index("c") * NUM_SUBCORES + lax.axis_index("s")
    per_worker = n_gathers * IDX_PER_GATHER
    base = wid * per_worker

    pltpu.sync_copy(idx_hbm.at[wid], idx_v)

    n_blocks = n_gathers // GATHERS_PER_BLOCK
    block_rows = GATHERS_PER_BLOCK * IDX_PER_GATHER

    @pl.loop(0, n_blocks)
    def _(b):
        copies = []
        for j in range(GATHERS_PER_BLOCK):
            copies.append(pltpu.async_copy(
                weight_hbm.at[idx_v.at[b * GATHERS_PER_BLOCK + j]],
                rows_v.at[pl.ds(j * IDX_PER_GATHER, IDX_PER_GATHER)],
                gsem))
        for c in copies:
            c.wait()
        pltpu.sync_copy(rows_v, out_hbm.at[pl.ds(base + b * block_rows,
                                                 block_rows)])


def kernel(input_, weight):
    batch, seq = input_.shape
    n = batch * seq
    dim = weight.shape[1]
    assert n % (NUM_WORKERS * IDX_PER_GATHER) == 0
    per_worker = n // NUM_WORKERS
    n_gathers = per_worker // IDX_PER_GATHER
    assert n_gathers % GATHERS_PER_BLOCK == 0

    idx = input_.astype(jnp.int32).reshape(NUM_WORKERS, n_gathers,
                                           IDX_PER_GATHER)

    mesh = plsc.VectorSubcoreMesh(core_axis_name="c", subcore_axis_name="s")
    block_rows = GATHERS_PER_BLOCK * IDX_PER_GATHER
    sc_gather = pl.kernel(
        functools.partial(_gather_kernel_body, n_gathers),
        out_type=jax.ShapeDtypeStruct((n, dim), weight.dtype),
        mesh=mesh,
        scratch_types=[
            pltpu.VMEM((n_gathers, IDX_PER_GATHER), jnp.int32),
            pltpu.VMEM((block_rows, dim), jnp.float32),
            pltpu.SemaphoreType.DMA,
            pltpu.SemaphoreType.DMA,
        ],
        compiler_params=pltpu.CompilerParams(use_tc_tiling_on_sc=False),
    )
    out = sc_gather(weight, idx)
    return out.reshape(batch, seq, dim)
